# baseline (speedup 1.0000x reference)
import functools

import jax
import jax.numpy as jnp
from jax.experimental import pallas as pl
from jax.experimental.pallas import tpu as pltpu

_NUM_EMBEDDINGS = 256
_LATENT = 128
_BETA = 4.0
_ROWS_PER_BLOCK = 256
_TOPK = 4


def _vq_block(x_ref, e_ref, out_ref, loss_ref):
    i = pl.program_id(0)
    x = x_ref[...]
    emb = e_ref[...]

    en = jnp.sum(emb * emb, axis=1)
    xe = jax.lax.dot_general(
        x, emb, (((1,), (1,)), ((), ())), preferred_element_type=jnp.float32,
        precision=jax.lax.Precision.HIGHEST)
    s = en[None, :] - 2.0 * xe

    iota = jax.lax.broadcasted_iota(jnp.int32, s.shape, 1)

    cand_idx = []
    work = s
    for _ in range(_TOPK):
        m = jnp.min(work, axis=1, keepdims=True)
        idx = jnp.min(jnp.where(work == m, iota, _NUM_EMBEDDINGS),
                      axis=1, keepdims=True)
        cand_idx.append(idx)
        work = jnp.where(iota == idx, jnp.inf, work)

    best_d = None
    for k in range(_TOPK):
        oh = (iota == cand_idx[k]).astype(jnp.float32)
        ek = jax.lax.dot_general(
            oh, emb, (((1,), (0,)), ((), ())),
            preferred_element_type=jnp.float32,
            precision=jax.lax.Precision.HIGHEST)
        diff = x - ek
        dk = jnp.sqrt(jnp.sum(jnp.square(diff), axis=1, keepdims=True))
        if best_d is None:
            best_d, best_i, best_e = dk, cand_idx[k], ek
        else:
            take = (dk < best_d) | ((dk == best_d) & (cand_idx[k] < best_i))
            best_d = jnp.where(take, dk, best_d)
            best_i = jnp.where(take, cand_idx[k], best_i)
            best_e = jnp.where(take, ek, best_e)

    out_ref[...] = x + (best_e - x)

    r = best_e - x
    part = jnp.sum(r * r).reshape(1, 1)

    @pl.when(i == 0)
    def _():
        loss_ref[...] = jnp.zeros((1, 1), jnp.float32)

    loss_ref[...] += part


@functools.partial(jax.jit, static_argnames=())
def kernel(inputs, embeddings):
    shape = inputs.shape
    n = shape[0] * shape[1] * shape[2]
    x = inputs.reshape(n, _LATENT)
    grid = n // _ROWS_PER_BLOCK

    out, loss_sum = pl.pallas_call(
        _vq_block,
        grid=(grid,),
        in_specs=[
            pl.BlockSpec((_ROWS_PER_BLOCK, _LATENT), lambda i: (i, 0)),
            pl.BlockSpec((_NUM_EMBEDDINGS, _LATENT), lambda i: (0, 0)),
        ],
        out_specs=[
            pl.BlockSpec((_ROWS_PER_BLOCK, _LATENT), lambda i: (i, 0)),
            pl.BlockSpec((1, 1), lambda i: (0, 0)),
        ],
        out_shape=[
            jax.ShapeDtypeStruct((n, _LATENT), jnp.float32),
            jax.ShapeDtypeStruct((1, 1), jnp.float32),
        ],
        compiler_params=pltpu.CompilerParams(
            dimension_semantics=("arbitrary",),
        ),
    )(x, embeddings)

    loss = loss_sum[0, 0] * ((1.0 + _BETA) / (n * _LATENT))
    return out.reshape(shape), loss
